# baseline (speedup 1.0000x reference)
import functools

import jax
import jax.numpy as jnp
from jax import lax
from jax.experimental import pallas as pl
from jax.experimental.pallas import tpu as pltpu
from jax.experimental.pallas import tpu_sc as plsc

N = 10000
E = 320000
D = 128
NQ = 4
QD = 32
H = 8
C = 16
NC = 2
NS = 16
CH = 128
NCH = 168
G = 24
NG = NCH // G
NBG = 3
NBS = 2
EPW = CH * NCH
EP = NS * EPW
RPT = 640
NR = NS * RPT
BR = 1000


def _sc_edge_pass(xl_hbm, xr_hbm, srcg_hbm, dstg_hbm, dsts_hbm, att_hbm,
                  msg_out, u_out,
                  att_v, srcg_big, dstg_big, dsts_big,
                  xl_rows0, xr_rows0, xl_rows1, xr_rows1, xl_rows2, xr_rows2,
                  stage_m0, stage_u0, stage_m1, stage_u1,
                  zero_m, zero_u, msg_acc, u_acc,
                  semg0, semg1, semg2, sems0, sems1):
    c = lax.axis_index("c")
    s = lax.axis_index("s")
    xl_rows = (xl_rows0, xl_rows1, xl_rows2)
    xr_rows = (xr_rows0, xr_rows1, xr_rows2)
    stage_m = (stage_m0, stage_m1)
    stage_u = (stage_u0, stage_u1)
    semg = (semg0, semg1, semg2)
    sems = (sems0, sems1)
    iota = lax.iota(jnp.int32, 16)
    rbase = s * RPT
    zv = jnp.zeros((16,), jnp.float32)

    def zrow(i, carry):
        for k in range(QD // 16):
            zero_m[i, pl.ds(k * 16, 16)] = zv
        zero_u[i, :] = zv
        return carry

    lax.fori_loop(0, 64, zrow, 0)

    def zero_own_slice():
        def zacc(i, carry):
            pltpu.sync_copy(zero_m, msg_acc.at[pl.ds(rbase + i * 64, 64)])
            pltpu.sync_copy(zero_u, u_acc.at[pl.ds(rbase + i * 64, 64)])
            return carry

        lax.fori_loop(0, RPT // 64, zacc, 0)

    pltpu.sync_copy(att_hbm, att_v)
    zero_own_slice()
    plsc.subcore_barrier()

    def issue_gather(gb, i):
        pltpu.async_copy(xl_hbm.at[srcg_big.at[i]], xl_rows[gb], semg[gb])
        pltpu.async_copy(xr_hbm.at[dstg_big.at[i]], xr_rows[gb], semg[gb])

    def wait_gather(gb):
        pltpu.make_async_copy(xl_hbm.at[srcg_big.at[0]], xl_rows[gb],
                              semg[gb]).wait()
        pltpu.make_async_copy(xr_hbm.at[dstg_big.at[0]], xr_rows[gb],
                              semg[gb]).wait()

    def wait_scatter(sb):
        pltpu.make_async_copy(stage_m[sb], msg_acc.at[dsts_big.at[0]],
                              sems[sb]).wait()
        pltpu.make_async_copy(stage_u[sb], u_acc.at[dsts_big.at[0]],
                              sems[sb]).wait()

    def issue_scatter(sb, i):
        pltpu.async_copy(stage_m[sb], msg_acc.at[dsts_big.at[i]], sems[sb],
                         add=True)
        pltpu.async_copy(stage_u[sb], u_acc.at[dsts_big.at[i]], sems[sb],
                         add=True)

    for ph in range(2):
        q = NC * c + ph
        att = [att_v[pl.ds(q * QD + h * 16, 16)] for h in range(2)]

        def compute(gb, sb):
            @plsc.parallel_loop(0, CH, unroll=8)
            def edge(e):
                usl = jnp.zeros((16,), jnp.float32)
                for h in range(2):
                    a = xl_rows[gb][e, pl.ds(h * 16, 16)]
                    bb = xr_rows[gb][e, pl.ds(h * 16, 16)]
                    v = a + bb
                    v = jnp.maximum(v, 0.2 * v)
                    lg = jnp.sum(v * att[h])
                    u = jnp.exp(jnp.broadcast_to(lg, (16,)))
                    stage_m[sb][e, pl.ds(h * 16, 16)] = u * a
                    usl = jnp.where(iota == h, u, usl)
                stage_u[sb][e, :] = usl

        def group(gl, carry):
            pltpu.sync_copy(srcg_hbm.at[(q * NS + s) * NG + gl], srcg_big)
            pltpu.sync_copy(dstg_hbm.at[(q * NS + s) * NG + gl], dstg_big)
            pltpu.sync_copy(dsts_hbm.at[s * NG + gl], dsts_big)
            issue_gather(0, 0)
            issue_gather(1, 1)

            def trip(t, tcarry):
                for b in range(2 * NBG):
                    i = 2 * NBG * t + b
                    gb = b % NBG
                    sb = b % NBS

                    @pl.when(i + 2 < G)
                    def _():
                        issue_gather((gb + 2) % NBG, i + 2)

                    wait_gather(gb)

                    if b >= NBS:
                        wait_scatter(sb)
                    else:
                        @pl.when(t > 0)
                        def _():
                            wait_scatter(sb)

                    compute(gb, sb)
                    issue_scatter(sb, i)
                return tcarry

            lax.fori_loop(0, G // (2 * NBG), trip, 0)
            for sb in range(NBS):
                wait_scatter(sb)
            return carry

        lax.fori_loop(0, NG, group, 0)
        plsc.subcore_barrier()
        pltpu.sync_copy(msg_acc.at[pl.ds(rbase, RPT)],
                        msg_out.at[c, ph, pl.ds(rbase, RPT)])
        pltpu.sync_copy(u_acc.at[pl.ds(rbase, RPT)],
                        u_out.at[c, ph, pl.ds(rbase, RPT)])
        if ph == 0:
            zero_own_slice()
            plsc.subcore_barrier()


@functools.cache
def _sc_call():
    return pl.kernel(
        _sc_edge_pass,
        out_type=[jax.ShapeDtypeStruct((NC, 2, NR, QD), jnp.float32),
                  jax.ShapeDtypeStruct((NC, 2, NR, 16), jnp.float32)],
        mesh=plsc.VectorSubcoreMesh(core_axis_name="c", subcore_axis_name="s"),
        compiler_params=pltpu.CompilerParams(needs_layout_passes=False,
                                             use_tc_tiling_on_sc=False),
        scratch_types=(
            [pltpu.VMEM((D,), jnp.float32),
             pltpu.VMEM((G, CH), jnp.int32),
             pltpu.VMEM((G, CH), jnp.int32),
             pltpu.VMEM((G, CH), jnp.int32)]
            + NBG * [pltpu.VMEM((CH, QD), jnp.float32),
                     pltpu.VMEM((CH, QD), jnp.float32)]
            + NBS * [pltpu.VMEM((CH, QD), jnp.float32),
                     pltpu.VMEM((CH, 16), jnp.float32)]
            + [pltpu.VMEM((64, QD), jnp.float32),
               pltpu.VMEM((64, 16), jnp.float32),
               pltpu.VMEM_SHARED((NR, QD), jnp.float32),
               pltpu.VMEM_SHARED((NR, 16), jnp.float32)]
            + (NBG + NBS) * [pltpu.SemaphoreType.DMA]
        ),
    )


def _mm_body(x_ref, w_ref, xl_ref, xr_ref):
    acc = jnp.dot(x_ref[...], w_ref[...], preferred_element_type=jnp.float32)
    for qq in range(NQ):
        xl_ref[qq] = acc[:, qq * QD:(qq + 1) * QD]
        xr_ref[qq] = acc[:, D + qq * QD:D + (qq + 1) * QD]


@functools.cache
def _mm_call():
    return pl.pallas_call(
        _mm_body,
        grid=(N // BR,),
        in_specs=[pl.BlockSpec((BR, D), lambda i: (i, 0)),
                  pl.BlockSpec((D, 2 * D), lambda i: (0, 0))],
        out_specs=[pl.BlockSpec((NQ, BR, QD), lambda i: (0, i, 0)),
                   pl.BlockSpec((NQ, BR, QD), lambda i: (0, i, 0))],
        out_shape=[jax.ShapeDtypeStruct((NQ, N, QD), jnp.float32),
                   jax.ShapeDtypeStruct((NQ, N, QD), jnp.float32)],
    )


def _norm_block(m_ref, u_ref, b_ref, bmat_ref):
    m = jnp.concatenate([m_ref[0, 0], m_ref[0, 1], m_ref[1, 0], m_ref[1, 1]],
                        axis=1)
    u8 = jnp.concatenate([u_ref[0, 0][:, :2], u_ref[0, 1][:, :2],
                          u_ref[1, 0][:, :2], u_ref[1, 1][:, :2]], axis=1)
    recip = 1.0 / (u8 + 1e-16)
    r128 = jnp.dot(recip, bmat_ref[...], preferred_element_type=jnp.float32)
    return m * r128 + b_ref[...]


def _comb1_body(m_ref, u_ref, b_ref, bmat_ref, w_ref, xl_ref, xr_ref):
    x1 = _norm_block(m_ref, u_ref, b_ref, bmat_ref)
    x1 = jnp.where(x1 > 0.0, x1, jnp.exp(x1) - 1.0)
    acc = jnp.dot(x1, w_ref[...], preferred_element_type=jnp.float32)
    for qq in range(NQ):
        xl_ref[qq] = acc[:, qq * QD:(qq + 1) * QD]
        xr_ref[qq] = acc[:, D + qq * QD:D + (qq + 1) * QD]


@functools.cache
def _comb1_call():
    return pl.pallas_call(
        _comb1_body,
        grid=(N // BR,),
        in_specs=[pl.BlockSpec((NC, 2, BR, QD), lambda i: (0, 0, i, 0)),
                  pl.BlockSpec((NC, 2, BR, 16), lambda i: (0, 0, i, 0)),
                  pl.BlockSpec((1, D), lambda i: (0, 0)),
                  pl.BlockSpec((H, D), lambda i: (0, 0)),
                  pl.BlockSpec((D, 2 * D), lambda i: (0, 0))],
        out_specs=[pl.BlockSpec((NQ, BR, QD), lambda i: (0, i, 0)),
                   pl.BlockSpec((NQ, BR, QD), lambda i: (0, i, 0))],
        out_shape=[jax.ShapeDtypeStruct((NQ, N, QD), jnp.float32),
                   jax.ShapeDtypeStruct((NQ, N, QD), jnp.float32)],
    )


def _comb2_body(m_ref, u_ref, b_ref, bmat_ref, h_ref, ls_ref):
    h = _norm_block(m_ref, u_ref, b_ref, bmat_ref)
    mx = jnp.max(h, axis=1, keepdims=True)
    lse = jnp.log(jnp.sum(jnp.exp(h - mx), axis=1, keepdims=True)) + mx
    h_ref[...] = h
    ls_ref[...] = h - lse


@functools.cache
def _comb2_call():
    return pl.pallas_call(
        _comb2_body,
        grid=(N // BR,),
        in_specs=[pl.BlockSpec((NC, 2, BR, QD), lambda i: (0, 0, i, 0)),
                  pl.BlockSpec((NC, 2, BR, 16), lambda i: (0, 0, i, 0)),
                  pl.BlockSpec((1, D), lambda i: (0, 0)),
                  pl.BlockSpec((H, D), lambda i: (0, 0))],
        out_specs=[pl.BlockSpec((BR, D), lambda i: (i, 0)),
                   pl.BlockSpec((BR, D), lambda i: (i, 0))],
        out_shape=[jax.ShapeDtypeStruct((N, D), jnp.float32),
                   jax.ShapeDtypeStruct((N, D), jnp.float32)],
    )


def kernel(x, edge_index, W_l1, W_r1, att1, b1, W_l2, W_r2, att2, b2):
    loops = jnp.arange(N, dtype=jnp.int32)
    src = jnp.concatenate([edge_index[0], loops])
    dst = jnp.concatenate([edge_index[1], loops])
    pad = EP - (E + N)
    zpad = jnp.zeros((pad,), jnp.int32)
    quarter_off = (N * jnp.arange(NQ, dtype=jnp.int32)).reshape(NQ, 1, 1, 1)
    srcg = (jnp.concatenate([src, zpad]).reshape(1, NS, NCH, CH)
            + quarter_off).reshape(NQ * NS * NG, G, CH)
    dstg = (jnp.concatenate([dst, zpad]).reshape(1, NS, NCH, CH)
            + quarter_off).reshape(NQ * NS * NG, G, CH)
    dsts = jnp.concatenate(
        [dst, jnp.full((pad,), N, jnp.int32)]).reshape(NS * NG, G, CH)

    bmat = jnp.repeat(jnp.eye(H, dtype=jnp.float32), C, axis=1)

    Wc1 = jnp.concatenate([W_l1, W_r1], axis=1)
    Wc2 = jnp.concatenate([W_l2, W_r2], axis=1)

    xl1, xr1 = _mm_call()(x, Wc1)
    msg1, u1 = _sc_call()(xl1.reshape(NQ * N, QD), xr1.reshape(NQ * N, QD),
                          srcg, dstg, dsts, att1.reshape(-1))
    xl2, xr2 = _comb1_call()(msg1, u1, b1.reshape(1, D), bmat, Wc2)
    msg2, u2 = _sc_call()(xl2.reshape(NQ * N, QD), xr2.reshape(NQ * N, QD),
                          srcg, dstg, dsts, att2.reshape(-1))
    h, ls = _comb2_call()(msg2, u2, b2.reshape(1, D), bmat)
    return (h, ls)


# speedup vs baseline: 1.0749x; 1.0749x over previous
import functools

import jax
import jax.numpy as jnp
from jax import lax
from jax.experimental import pallas as pl
from jax.experimental.pallas import tpu as pltpu
from jax.experimental.pallas import tpu_sc as plsc

N = 10000
E = 320000
D = 128
HD = 64
H = 8
C = 16
NC = 2
NS = 16
CH = 128
NCH = 168
G = 24
NG = NCH // G
NBG = 2
NBS = 2
EPW = CH * NCH
EP = NS * EPW
RPT = 640
NR = NS * RPT
BR = 1000


def _sc_edge_pass(xl_hbm, xr_hbm, srcg_hbm, dstg_hbm, dsts_hbm, att_hbm,
                  msg_out, u_out,
                  att_v, srcg_big, dstg_big, dsts_big,
                  xl_rows0, xr_rows0, xl_rows1, xr_rows1,
                  stage_m0, stage_u0, stage_m1, stage_u1,
                  zero_m, zero_u, msg_acc, u_acc,
                  semg0, semg1, sems0, sems1):
    c = lax.axis_index("c")
    s = lax.axis_index("s")
    xl_rows = (xl_rows0, xl_rows1)
    xr_rows = (xr_rows0, xr_rows1)
    stage_m = (stage_m0, stage_m1)
    stage_u = (stage_u0, stage_u1)
    semg = (semg0, semg1)
    sems = (sems0, sems1)
    iota = lax.iota(jnp.int32, 16)
    rbase = s * RPT
    zv = jnp.zeros((16,), jnp.float32)

    def zrow(i, carry):
        for k in range(HD // 16):
            zero_m[i, pl.ds(k * 16, 16)] = zv
        zero_u[i, :] = zv
        return carry

    lax.fori_loop(0, 64, zrow, 0)

    def zacc(i, carry):
        pltpu.sync_copy(zero_m, msg_acc.at[pl.ds(rbase + i * 64, 64)])
        pltpu.sync_copy(zero_u, u_acc.at[pl.ds(rbase + i * 64, 64)])
        return carry

    lax.fori_loop(0, RPT // 64, zacc, 0)
    pltpu.sync_copy(att_hbm, att_v)
    plsc.subcore_barrier()

    att = [att_v[pl.ds(c * HD + h * 16, 16)] for h in range(H // NC)]

    def issue_gather(gb, i):
        pltpu.async_copy(xl_hbm.at[srcg_big.at[i]], xl_rows[gb], semg[gb])
        pltpu.async_copy(xr_hbm.at[dstg_big.at[i]], xr_rows[gb], semg[gb])

    def wait_gather(gb):
        pltpu.make_async_copy(xl_hbm.at[srcg_big.at[0]], xl_rows[gb],
                              semg[gb]).wait()
        pltpu.make_async_copy(xr_hbm.at[dstg_big.at[0]], xr_rows[gb],
                              semg[gb]).wait()

    def wait_scatter(sb):
        pltpu.make_async_copy(stage_m[sb], msg_acc.at[dsts_big.at[0]],
                              sems[sb]).wait()
        pltpu.make_async_copy(stage_u[sb], u_acc.at[dsts_big.at[0]],
                              sems[sb]).wait()

    def issue_scatter(sb, i):
        pltpu.async_copy(stage_m[sb], msg_acc.at[dsts_big.at[i]], sems[sb],
                         add=True)
        pltpu.async_copy(stage_u[sb], u_acc.at[dsts_big.at[i]], sems[sb],
                         add=True)

    def compute(gb, sb):
        @plsc.parallel_loop(0, CH, unroll=8)
        def edge(e):
            usl = jnp.zeros((16,), jnp.float32)
            for h in range(H // NC):
                a = xl_rows[gb][e, pl.ds(h * 16, 16)]
                bb = xr_rows[gb][e, pl.ds(h * 16, 16)]
                v = a + bb
                v = jnp.maximum(v, 0.2 * v)
                lg = jnp.sum(v * att[h])
                u = jnp.exp(jnp.broadcast_to(lg, (16,)))
                stage_m[sb][e, pl.ds(h * 16, 16)] = u * a
                usl = jnp.where(iota == h, u, usl)
            stage_u[sb][e, :] = usl

    def group(gl, carry):
        pltpu.sync_copy(srcg_hbm.at[(c * NS + s) * NG + gl], srcg_big)
        pltpu.sync_copy(dstg_hbm.at[(c * NS + s) * NG + gl], dstg_big)
        pltpu.sync_copy(dsts_hbm.at[s * NG + gl], dsts_big)
        issue_gather(0, 0)
        issue_gather(1, 1)

        def trip(t, tcarry):
            for b in range(2 * NBG):
                i = 2 * NBG * t + b
                gb = b % NBG
                sb = b % NBS

                wait_gather(gb)

                if b >= NBS:
                    wait_scatter(sb)
                else:
                    @pl.when(t > 0)
                    def _():
                        wait_scatter(sb)

                compute(gb, sb)

                @pl.when(i + 2 < G)
                def _():
                    issue_gather(gb, i + 2)

                issue_scatter(sb, i)
            return tcarry

        lax.fori_loop(0, G // (2 * NBG), trip, 0)
        for sb in range(NBS):
            wait_scatter(sb)
        return carry

    lax.fori_loop(0, NG, group, 0)
    plsc.subcore_barrier()
    pltpu.sync_copy(msg_acc.at[pl.ds(rbase, RPT)],
                    msg_out.at[c, pl.ds(rbase, RPT)])
    pltpu.sync_copy(u_acc.at[pl.ds(rbase, RPT)],
                    u_out.at[c, pl.ds(rbase, RPT)])


@functools.cache
def _sc_call():
    return pl.kernel(
        _sc_edge_pass,
        out_type=[jax.ShapeDtypeStruct((NC, NR, HD), jnp.float32),
                  jax.ShapeDtypeStruct((NC, NR, 16), jnp.float32)],
        mesh=plsc.VectorSubcoreMesh(core_axis_name="c", subcore_axis_name="s"),
        compiler_params=pltpu.CompilerParams(needs_layout_passes=False,
                                             use_tc_tiling_on_sc=False),
        scratch_types=(
            [pltpu.VMEM((D,), jnp.float32),
             pltpu.VMEM((G, CH), jnp.int32),
             pltpu.VMEM((G, CH), jnp.int32),
             pltpu.VMEM((G, CH), jnp.int32)]
            + NBG * [pltpu.VMEM((CH, HD), jnp.float32),
                     pltpu.VMEM((CH, HD), jnp.float32)]
            + NBS * [pltpu.VMEM((CH, HD), jnp.float32),
                     pltpu.VMEM((CH, 16), jnp.float32)]
            + [pltpu.VMEM((64, HD), jnp.float32),
               pltpu.VMEM((64, 16), jnp.float32),
               pltpu.VMEM_SHARED((NR, HD), jnp.float32),
               pltpu.VMEM_SHARED((NR, 16), jnp.float32)]
            + (NBG + NBS) * [pltpu.SemaphoreType.DMA]
        ),
    )


def _mm_body(x_ref, w_ref, xl_ref, xr_ref):
    acc = jnp.dot(x_ref[...], w_ref[...], preferred_element_type=jnp.float32)
    xl_ref[0] = acc[:, 0 * HD:1 * HD]
    xl_ref[1] = acc[:, 1 * HD:2 * HD]
    xr_ref[0] = acc[:, 2 * HD:3 * HD]
    xr_ref[1] = acc[:, 3 * HD:4 * HD]


@functools.cache
def _mm_call():
    return pl.pallas_call(
        _mm_body,
        grid=(N // BR,),
        in_specs=[pl.BlockSpec((BR, D), lambda i: (i, 0)),
                  pl.BlockSpec((D, 2 * D), lambda i: (0, 0))],
        out_specs=[pl.BlockSpec((NC, BR, HD), lambda i: (0, i, 0)),
                   pl.BlockSpec((NC, BR, HD), lambda i: (0, i, 0))],
        out_shape=[jax.ShapeDtypeStruct((NC, N, HD), jnp.float32),
                   jax.ShapeDtypeStruct((NC, N, HD), jnp.float32)],
    )


def _norm_block(m_ref, u_ref, b_ref, bmat_ref):
    m = jnp.concatenate([m_ref[0], m_ref[1]], axis=1)
    u8 = jnp.concatenate([u_ref[0][:, :H // NC], u_ref[1][:, :H // NC]],
                         axis=1)
    recip = 1.0 / (u8 + 1e-16)
    r128 = jnp.dot(recip, bmat_ref[...], preferred_element_type=jnp.float32)
    return m * r128 + b_ref[...]


def _comb1_body(m_ref, u_ref, b_ref, bmat_ref, w_ref, xl_ref, xr_ref):
    x1 = _norm_block(m_ref, u_ref, b_ref, bmat_ref)
    x1 = jnp.where(x1 > 0.0, x1, jnp.exp(x1) - 1.0)
    acc = jnp.dot(x1, w_ref[...], preferred_element_type=jnp.float32)
    xl_ref[0] = acc[:, 0 * HD:1 * HD]
    xl_ref[1] = acc[:, 1 * HD:2 * HD]
    xr_ref[0] = acc[:, 2 * HD:3 * HD]
    xr_ref[1] = acc[:, 3 * HD:4 * HD]


@functools.cache
def _comb1_call():
    return pl.pallas_call(
        _comb1_body,
        grid=(N // BR,),
        in_specs=[pl.BlockSpec((NC, BR, HD), lambda i: (0, i, 0)),
                  pl.BlockSpec((NC, BR, 16), lambda i: (0, i, 0)),
                  pl.BlockSpec((1, D), lambda i: (0, 0)),
                  pl.BlockSpec((H, D), lambda i: (0, 0)),
                  pl.BlockSpec((D, 2 * D), lambda i: (0, 0))],
        out_specs=[pl.BlockSpec((NC, BR, HD), lambda i: (0, i, 0)),
                   pl.BlockSpec((NC, BR, HD), lambda i: (0, i, 0))],
        out_shape=[jax.ShapeDtypeStruct((NC, N, HD), jnp.float32),
                   jax.ShapeDtypeStruct((NC, N, HD), jnp.float32)],
    )


def _comb2_body(m_ref, u_ref, b_ref, bmat_ref, h_ref, ls_ref):
    h = _norm_block(m_ref, u_ref, b_ref, bmat_ref)
    mx = jnp.max(h, axis=1, keepdims=True)
    lse = jnp.log(jnp.sum(jnp.exp(h - mx), axis=1, keepdims=True)) + mx
    h_ref[...] = h
    ls_ref[...] = h - lse


@functools.cache
def _comb2_call():
    return pl.pallas_call(
        _comb2_body,
        grid=(N // BR,),
        in_specs=[pl.BlockSpec((NC, BR, HD), lambda i: (0, i, 0)),
                  pl.BlockSpec((NC, BR, 16), lambda i: (0, i, 0)),
                  pl.BlockSpec((1, D), lambda i: (0, 0)),
                  pl.BlockSpec((H, D), lambda i: (0, 0))],
        out_specs=[pl.BlockSpec((BR, D), lambda i: (i, 0)),
                   pl.BlockSpec((BR, D), lambda i: (i, 0))],
        out_shape=[jax.ShapeDtypeStruct((N, D), jnp.float32),
                   jax.ShapeDtypeStruct((N, D), jnp.float32)],
    )


def kernel(x, edge_index, W_l1, W_r1, att1, b1, W_l2, W_r2, att2, b2):
    loops = jnp.arange(N, dtype=jnp.int32)
    src = jnp.concatenate([edge_index[0], loops])
    dst = jnp.concatenate([edge_index[1], loops])
    pad = EP - (E + N)
    zpad = jnp.zeros((pad,), jnp.int32)
    core_off = jnp.array([0, N], jnp.int32).reshape(NC, 1, 1, 1)
    srcg = (jnp.concatenate([src, zpad]).reshape(1, NS, NCH, CH)
            + core_off).reshape(NC * NS * NG, G, CH)
    dstg = (jnp.concatenate([dst, zpad]).reshape(1, NS, NCH, CH)
            + core_off).reshape(NC * NS * NG, G, CH)
    dsts = jnp.concatenate(
        [dst, jnp.full((pad,), N, jnp.int32)]).reshape(NS * NG, G, CH)

    bmat = jnp.repeat(jnp.eye(H, dtype=jnp.float32), C, axis=1)

    Wc1 = jnp.concatenate([W_l1, W_r1], axis=1)
    Wc2 = jnp.concatenate([W_l2, W_r2], axis=1)

    xl1, xr1 = _mm_call()(x, Wc1)
    msg1, u1 = _sc_call()(xl1.reshape(NC * N, HD), xr1.reshape(NC * N, HD),
                          srcg, dstg, dsts, att1.reshape(-1))
    xl2, xr2 = _comb1_call()(msg1, u1, b1.reshape(1, D), bmat, Wc2)
    msg2, u2 = _sc_call()(xl2.reshape(NC * N, HD), xr2.reshape(NC * N, HD),
                          srcg, dstg, dsts, att2.reshape(-1))
    h, ls = _comb2_call()(msg2, u2, b2.reshape(1, D), bmat)
    return (h, ls)
